# baseline (speedup 1.0000x reference)
import functools

import jax
import jax.numpy as jnp
from jax import lax
from jax.experimental import pallas as pl
from jax.experimental.pallas import tpu as pltpu
from jax.experimental.pallas import tpu_sc as plsc

B = 16384
D = 32
T = 4
NE = 100000
NC = 2
NS = 16
NW = NC * NS
BPW = B // NW
CHUNK = 128
NIDX = BPW // CHUNK
NMETA = BPW * T
NMCH = NMETA // CHUNK

_mesh = plsc.VectorSubcoreMesh(core_axis_name="c", subcore_axis_name="s")


@functools.partial(
    pl.kernel,
    mesh=_mesh,
    out_type=jax.ShapeDtypeStruct((B, D), jnp.float32),
    compiler_params=pltpu.CompilerParams(use_tc_tiling_on_sc=False),
    scratch_types=[
        pltpu.VMEM((NIDX, CHUNK), jnp.int32),
        pltpu.VMEM((NMCH, CHUNK), jnp.int32),
        pltpu.VMEM((NMCH, CHUNK), jnp.int32),
        pltpu.VMEM((NMCH, CHUNK), jnp.float32),
        pltpu.VMEM((NMETA, D), jnp.float32),
        pltpu.VMEM((BPW, D), jnp.float32),
        pltpu.SemaphoreType.DMA,
        pltpu.SemaphoreType.DMA,
        pltpu.SemaphoreType.DMA,
    ],
)
def _sc_meta_sum(e2_hbm, tm_hbm, mif_hbm, mwf_hbm, ws_hbm,
                 idx_v, fidx_v, mi_v, mw_v, meta_v, ws_v,
                 sem_i, sem_w, sem_m):
    wid = lax.axis_index("s") * NC + lax.axis_index("c")
    base = wid * BPW

    pltpu.sync_copy(e2_hbm.at[pl.ds(wid * NIDX, NIDX)], idx_v)

    def flat_body(k, _):
        v = idx_v[k // 8, pl.ds((k % 8) * 16, 16)]
        for t in range(T):
            fidx_v[t * NIDX + k // 8, pl.ds((k % 8) * 16, 16)] = v + t * NE
        return 0

    lax.fori_loop(0, BPW // 16, flat_body, 0)

    mi_d, mw_d = [], []
    for j in range(NMCH):
        mi_d.append(pltpu.async_copy(
            mif_hbm.at[fidx_v.at[j]], mi_v.at[j], sem_i))
    for j in range(NMCH):
        mw_d.append(pltpu.async_copy(
            mwf_hbm.at[fidx_v.at[j]], mw_v.at[j], sem_w))
    for d in mi_d:
        d.wait()

    mt_d = []
    for j in range(NMCH):
        mt_d.append(pltpu.async_copy(
            tm_hbm.at[mi_v.at[j]], meta_v.at[pl.ds(j * CHUNK, CHUNK)], sem_m))
    for d in mw_d:
        d.wait()
    for d in mt_d:
        d.wait()

    def group_body(g, _):
        r = g // 8
        cb = (g % 8) * 16
        wv = [mw_v[t * NIDX + r, pl.ds(cb, 16)] for t in range(T)]
        for l in range(16):
            b = g * 16 + l
            w = [jnp.full((16,), wv[t][l]) for t in range(T)]
            for h in range(D // 16):
                sl = pl.ds(h * 16, 16)
                acc = w[0] * meta_v[b, sl]
                for t in range(1, T):
                    acc = acc + w[t] * meta_v[t * BPW + b, sl]
                ws_v[b, sl] = acc
        return 0

    lax.fori_loop(0, BPW // 16, group_body, 0)

    pltpu.sync_copy(ws_v, ws_hbm.at[pl.ds(base, BPW)])


@functools.partial(
    pl.kernel,
    mesh=_mesh,
    out_type=jax.ShapeDtypeStruct((B, D), jnp.float32),
    compiler_params=pltpu.CompilerParams(use_tc_tiling_on_sc=False),
    scratch_types=[
        pltpu.VMEM((NIDX, CHUNK), jnp.int32),
        pltpu.VMEM((BPW, CHUNK), jnp.float32),
        pltpu.VMEM((BPW, D), jnp.float32),
        pltpu.SemaphoreType.DMA,
        pltpu.SemaphoreType.DMA,
    ],
)
def _sc_elem_add(e2_hbm, te_hbm, ws_hbm, out_hbm,
                 idx_v, elem_v, ws_v, sem_e, sem_s):
    wid = lax.axis_index("s") * NC + lax.axis_index("c")
    base = wid * BPW

    pltpu.sync_copy(e2_hbm.at[pl.ds(wid * NIDX, NIDX)], idx_v)
    ws_d = pltpu.async_copy(ws_hbm.at[pl.ds(base, BPW)], ws_v, sem_s)
    el_d = []
    for j in range(NIDX):
        el_d.append(pltpu.async_copy(
            te_hbm.at[idx_v.at[j]], elem_v.at[pl.ds(j * CHUNK, CHUNK)], sem_e))
    for d in el_d:
        d.wait()
    ws_d.wait()

    scale = jnp.float32(0.2)

    def row_body(b, _):
        for h in range(D // 16):
            sl = pl.ds(h * 16, 16)
            ws_v[b, sl] = (elem_v[b, sl] + ws_v[b, sl]) * scale
        return 0

    lax.fori_loop(0, BPW, row_body, 0)

    pltpu.sync_copy(ws_v, out_hbm.at[pl.ds(base, BPW)])


def kernel(element_indicies, table_element, table_meta, meta_indicies,
           meta_weights):
    e2 = element_indicies.reshape(B // CHUNK, CHUNK)
    wsum = _sc_meta_sum(e2, table_meta,
                        meta_indicies.T.reshape(-1),
                        meta_weights.T.reshape(-1))
    te_pad = jnp.pad(table_element, ((0, 0), (0, CHUNK - D)))
    return _sc_elem_add(e2, te_pad, wsum)


# speedup vs baseline: 1.1907x; 1.1907x over previous
import functools

import jax
import jax.numpy as jnp
from jax import lax
from jax.experimental import pallas as pl
from jax.experimental.pallas import tpu as pltpu
from jax.experimental.pallas import tpu_sc as plsc

B = 16384
D = 32
T = 4
NE = 100000
NM = 1000
NC = 2
NS = 16
NW = NC * NS
BPW = B // NW
CHUNK = 128
NIDX = BPW // CHUNK
NMETA = BPW * T
NMCH = NMETA // CHUNK
FSH = NE * T // NS

_mesh = plsc.VectorSubcoreMesh(core_axis_name="c", subcore_axis_name="s")


@functools.partial(
    pl.kernel,
    mesh=_mesh,
    out_type=jax.ShapeDtypeStruct((B, D), jnp.float32),
    compiler_params=pltpu.CompilerParams(use_tc_tiling_on_sc=False),
    scratch_types=[
        pltpu.VMEM((NIDX, CHUNK), jnp.int32),
        pltpu.VMEM((NMCH, CHUNK), jnp.int32),
        pltpu.VMEM((NMCH, CHUNK), jnp.int32),
        pltpu.VMEM((NMCH, CHUNK), jnp.float32),
        pltpu.VMEM((NMETA, D), jnp.float32),
        pltpu.VMEM((BPW, D), jnp.float32),
        pltpu.VMEM_SHARED((NE * T,), jnp.int32),
        pltpu.VMEM_SHARED((NM, D), jnp.float32),
        pltpu.SemaphoreType.DMA,
        pltpu.SemaphoreType.DMA,
        pltpu.SemaphoreType.DMA,
    ],
)
def _sc_meta_sum(e2_hbm, tm_hbm, mif_hbm, mwf_hbm, ws_hbm,
                 idx_v, fidx_v, mi_v, mw_v, meta_v, ws_v,
                 sh_mi, sh_tm,
                 sem_i, sem_w, sem_m):
    sid = lax.axis_index("s")
    wid = sid * NC + lax.axis_index("c")
    base = wid * BPW

    sh = sid * FSH
    pltpu.sync_copy(mif_hbm.at[pl.ds(sh, FSH)], sh_mi.at[pl.ds(sh, FSH)])

    @pl.when(sid == 0)
    def _():
        pltpu.sync_copy(tm_hbm, sh_tm)

    pltpu.sync_copy(e2_hbm.at[pl.ds(wid * NIDX, NIDX)], idx_v)

    def flat_body(k, _):
        v = idx_v[k // 8, pl.ds((k % 8) * 16, 16)]
        for t in range(T):
            fidx_v[t * NIDX + k // 8, pl.ds((k % 8) * 16, 16)] = v + t * NE
        return 0

    lax.fori_loop(0, BPW // 16, flat_body, 0)

    plsc.subcore_barrier()

    mi_d, mw_d = [], []
    for j in range(NMCH):
        mi_d.append(pltpu.async_copy(
            sh_mi.at[fidx_v.at[j]], mi_v.at[j], sem_i))
    for j in range(NMCH):
        mw_d.append(pltpu.async_copy(
            mwf_hbm.at[fidx_v.at[j]], mw_v.at[j], sem_w))
    for d in mi_d:
        d.wait()

    mt_d = []
    for j in range(NMCH):
        mt_d.append(pltpu.async_copy(
            sh_tm.at[mi_v.at[j]], meta_v.at[pl.ds(j * CHUNK, CHUNK)], sem_m))
    for d in mw_d:
        d.wait()
    for d in mt_d:
        d.wait()

    def group_body(g, _):
        r = g // 8
        cb = (g % 8) * 16
        wv = [mw_v[t * NIDX + r, pl.ds(cb, 16)] for t in range(T)]
        for l in range(16):
            b = g * 16 + l
            w = [jnp.full((16,), wv[t][l]) for t in range(T)]
            for h in range(D // 16):
                sl = pl.ds(h * 16, 16)
                acc = w[0] * meta_v[b, sl]
                for t in range(1, T):
                    acc = acc + w[t] * meta_v[t * BPW + b, sl]
                ws_v[b, sl] = acc
        return 0

    lax.fori_loop(0, BPW // 16, group_body, 0)

    pltpu.sync_copy(ws_v, ws_hbm.at[pl.ds(base, BPW)])


@functools.partial(
    pl.kernel,
    mesh=_mesh,
    out_type=jax.ShapeDtypeStruct((B, D), jnp.float32),
    compiler_params=pltpu.CompilerParams(use_tc_tiling_on_sc=False),
    scratch_types=[
        pltpu.VMEM((NIDX, CHUNK), jnp.int32),
        pltpu.VMEM((BPW, D), jnp.float32),
        pltpu.VMEM((BPW, D), jnp.float32),
        pltpu.SemaphoreType.DMA,
        pltpu.SemaphoreType.DMA,
    ],
)
def _sc_elem_add(e2_hbm, te_hbm, ws_hbm, out_hbm,
                 idx_v, elem_v, ws_v, sem_e, sem_s):
    wid = lax.axis_index("s") * NC + lax.axis_index("c")
    base = wid * BPW

    pltpu.sync_copy(e2_hbm.at[pl.ds(wid * NIDX, NIDX)], idx_v)
    ws_d = pltpu.async_copy(ws_hbm.at[pl.ds(base, BPW)], ws_v, sem_s)
    el_d = []
    for j in range(NIDX):
        el_d.append(pltpu.async_copy(
            te_hbm.at[idx_v.at[j]], elem_v.at[pl.ds(j * CHUNK, CHUNK)], sem_e))
    for d in el_d:
        d.wait()
    ws_d.wait()

    scale = jnp.float32(0.2)

    def row_body(b, _):
        for h in range(D // 16):
            sl = pl.ds(h * 16, 16)
            ws_v[b, sl] = (elem_v[b, sl] + ws_v[b, sl]) * scale
        return 0

    lax.fori_loop(0, BPW, row_body, 0)

    pltpu.sync_copy(ws_v, out_hbm.at[pl.ds(base, BPW)])


def kernel(element_indicies, table_element, table_meta, meta_indicies,
           meta_weights):
    e2 = element_indicies.reshape(B // CHUNK, CHUNK)
    wsum = _sc_meta_sum(e2, table_meta,
                        meta_indicies.T.reshape(-1),
                        meta_weights.T.reshape(-1))
    return _sc_elem_add(e2, table_element, wsum)


# speedup vs baseline: 1.2588x; 1.0572x over previous
import functools

import jax
import jax.numpy as jnp
from jax import lax
from jax.experimental import pallas as pl
from jax.experimental.pallas import tpu as pltpu
from jax.experimental.pallas import tpu_sc as plsc

B = 16384
D = 32
T = 4
NE = 100000
NM = 1000
NC = 2
NS = 16
NW = NC * NS
BPW = B // NW
CHUNK = 128
NIDX = BPW // CHUNK
NMETA = BPW * T
NMCH = NMETA // CHUNK
FSH = NE * T // NS

_mesh = plsc.VectorSubcoreMesh(core_axis_name="c", subcore_axis_name="s")


@functools.partial(
    pl.kernel,
    mesh=_mesh,
    out_type=jax.ShapeDtypeStruct((B, D), jnp.float32),
    compiler_params=pltpu.CompilerParams(use_tc_tiling_on_sc=False),
    scratch_types=[
        pltpu.VMEM((NIDX, CHUNK), jnp.int32),
        pltpu.VMEM((NMCH, CHUNK), jnp.int32),
        pltpu.VMEM((NMCH, CHUNK), jnp.int32),
        pltpu.VMEM((NMCH, CHUNK), jnp.float32),
        pltpu.VMEM((NMETA, D), jnp.float32),
        pltpu.VMEM((BPW, D), jnp.float32),
        pltpu.VMEM_SHARED((NE * T,), jnp.int32),
        pltpu.VMEM_SHARED((NM, D), jnp.float32),
        pltpu.SemaphoreType.DMA,
        pltpu.SemaphoreType.DMA,
        pltpu.SemaphoreType.DMA,
    ],
)
def _sc_meta_sum(e2_hbm, tm_hbm, mif_hbm, mwf_hbm, ws_hbm,
                 idx_v, fidx_v, mi_v, mw_v, meta_v, ws_v,
                 sh_mi, sh_tm,
                 sem_i, sem_w, sem_m):
    sid = lax.axis_index("s")
    wid = sid * NC + lax.axis_index("c")
    base = wid * BPW

    sh = sid * FSH
    pltpu.sync_copy(mif_hbm.at[pl.ds(sh, FSH)], sh_mi.at[pl.ds(sh, FSH)])

    @pl.when(sid == 0)
    def _():
        pltpu.sync_copy(tm_hbm, sh_tm)

    pltpu.sync_copy(e2_hbm.at[pl.ds(wid * NIDX, NIDX)], idx_v)

    def flat_body(k, _):
        v = idx_v[k // 8, pl.ds((k % 8) * 16, 16)]
        for t in range(T):
            fidx_v[t * NIDX + k // 8, pl.ds((k % 8) * 16, 16)] = v + t * NE
        return 0

    lax.fori_loop(0, BPW // 16, flat_body, 0)

    plsc.subcore_barrier()

    mi_d, mw_d = [], []
    for j in range(NMCH):
        mi_d.append(pltpu.async_copy(
            sh_mi.at[fidx_v.at[j]], mi_v.at[j], sem_i))
    for j in range(NMCH):
        mw_d.append(pltpu.async_copy(
            mwf_hbm.at[fidx_v.at[j]], mw_v.at[j], sem_w))
    for d in mi_d:
        d.wait()

    mt_d = []
    for j in range(NMCH):
        mt_d.append(pltpu.async_copy(
            sh_tm.at[mi_v.at[j]], meta_v.at[pl.ds(j * CHUNK, CHUNK)], sem_m))
    for d in mw_d:
        d.wait()
    for d in mt_d:
        d.wait()

    def group_body(g, _):
        r = g // 8
        cb = (g % 8) * 16
        wv = [mw_v[t * NIDX + r, pl.ds(cb, 16)] for t in range(T)]
        for l in range(16):
            b = g * 16 + l
            w = [jnp.full((16,), wv[t][l]) for t in range(T)]
            for h in range(D // 16):
                sl = pl.ds(h * 16, 16)
                acc = w[0] * meta_v[b, sl]
                for t in range(1, T):
                    acc = acc + w[t] * meta_v[t * BPW + b, sl]
                ws_v[b, sl] = acc
        return 0

    lax.fori_loop(0, BPW // 16, group_body, 0)

    pltpu.sync_copy(ws_v, ws_hbm.at[pl.ds(base, BPW)])


@functools.partial(
    pl.kernel,
    mesh=_mesh,
    out_type=jax.ShapeDtypeStruct((D, B), jnp.float32),
    compiler_params=pltpu.CompilerParams(use_tc_tiling_on_sc=False),
    scratch_types=[
        pltpu.VMEM((NIDX, CHUNK), jnp.int32),
        pltpu.VMEM((D * NIDX, CHUNK), jnp.int32),
        pltpu.VMEM((D, BPW), jnp.float32),
        pltpu.VMEM((D, BPW), jnp.float32),
        pltpu.SemaphoreType.DMA,
        pltpu.SemaphoreType.DMA,
        pltpu.SemaphoreType.DMA,
    ],
)
def _sc_elem_add(e2_hbm, tef_hbm, wsT_hbm, out_hbm,
                 idx_v, fidx_v, el_v, ws_v, sem_e, sem_s, sem_o):
    wid = lax.axis_index("s") * NC + lax.axis_index("c")
    base = wid * BPW

    pltpu.sync_copy(e2_hbm.at[pl.ds(wid * NIDX, NIDX)], idx_v)
    ws_d = []
    for d in range(D):
        ws_d.append(pltpu.async_copy(
            wsT_hbm.at[d, pl.ds(base, BPW)], ws_v.at[d], sem_s))

    def flat_body(k, _):
        v = idx_v[k // 8, pl.ds((k % 8) * 16, 16)]
        for d in range(D):
            fidx_v[d * NIDX + k // 8, pl.ds((k % 8) * 16, 16)] = v + d * NE
        return 0

    lax.fori_loop(0, BPW // 16, flat_body, 0)

    el_d = []
    for d in range(D):
        for c in range(NIDX):
            el_d.append(pltpu.async_copy(
                tef_hbm.at[fidx_v.at[d * NIDX + c]],
                el_v.at[d, pl.ds(c * CHUNK, CHUNK)], sem_e))
    for dd in el_d:
        dd.wait()
    for dd in ws_d:
        dd.wait()

    scale = jnp.float32(0.2)

    def comb_body(r, _):
        for h in range(8):
            sl = pl.ds(128 * (r % NIDX) + 16 * h, 16)
            d = r // NIDX
            ws_v[d, sl] = (el_v[d, sl] + ws_v[d, sl]) * scale
        return 0

    lax.fori_loop(0, D * NIDX, comb_body, 0)

    out_d = []
    for d in range(D):
        out_d.append(pltpu.async_copy(
            ws_v.at[d], out_hbm.at[d, pl.ds(base, BPW)], sem_o))
    for dd in out_d:
        dd.wait()


def kernel(element_indicies, table_element, table_meta, meta_indicies,
           meta_weights):
    e2 = element_indicies.reshape(B // CHUNK, CHUNK)
    wsum = _sc_meta_sum(e2, table_meta,
                        meta_indicies.T.reshape(-1),
                        meta_weights.T.reshape(-1))
    outT = _sc_elem_add(e2, table_element.T.reshape(-1), wsum.T)
    return outT.T


# speedup vs baseline: 1.2871x; 1.0224x over previous
import functools

import jax
import jax.numpy as jnp
from jax import lax
from jax.experimental import pallas as pl
from jax.experimental.pallas import tpu as pltpu
from jax.experimental.pallas import tpu_sc as plsc

B = 16384
D = 32
T = 4
NE = 100000
NM = 1000
NC = 2
NS = 16
NW = NC * NS
BPW = B // NW
CHUNK = 128
NIDX = BPW // CHUNK
NMETA = BPW * T
NMCH = NMETA // CHUNK
FSH = NE * T // NS

_mesh = plsc.VectorSubcoreMesh(core_axis_name="c", subcore_axis_name="s")


@functools.partial(
    pl.kernel,
    mesh=_mesh,
    out_type=jax.ShapeDtypeStruct((D, B), jnp.float32),
    compiler_params=pltpu.CompilerParams(use_tc_tiling_on_sc=False),
    scratch_types=[
        pltpu.VMEM((NIDX, CHUNK), jnp.int32),
        pltpu.VMEM((NMCH, CHUNK), jnp.int32),
        pltpu.VMEM((NMCH, CHUNK), jnp.int32),
        pltpu.VMEM((NMCH, CHUNK), jnp.float32),
        pltpu.VMEM((NMETA, D), jnp.float32),
        pltpu.VMEM((BPW * D,), jnp.float32),
        pltpu.VMEM((D, CHUNK), jnp.int32),
        pltpu.VMEM((D, BPW), jnp.float32),
        pltpu.VMEM_SHARED((NM, D), jnp.float32),
        pltpu.VMEM_SHARED((NS * CHUNK * D,), jnp.float32),
        pltpu.SemaphoreType.DMA,
        pltpu.SemaphoreType.DMA,
        pltpu.SemaphoreType.DMA,
    ],
)
def _sc_meta_sum(e2_hbm, tm_hbm, mif_hbm, mwf_hbm, ws_hbm,
                 idx_v, fidx_v, mi_v, mw_v, meta_v, ws_v, tidx_v, wsT_v,
                 sh_tm, sh_ws,
                 sem_i, sem_w, sem_m):
    sid = lax.axis_index("s")
    wid = sid * NC + lax.axis_index("c")
    base = wid * BPW

    @pl.when(sid == 0)
    def _():
        pltpu.sync_copy(tm_hbm, sh_tm)

    pltpu.sync_copy(e2_hbm.at[pl.ds(wid * NIDX, NIDX)], idx_v)

    def flat_body(k, _):
        v = idx_v[k // 8, pl.ds((k % 8) * 16, 16)]
        for t in range(T):
            fidx_v[t * NIDX + k // 8, pl.ds((k % 8) * 16, 16)] = v + t * NE
        return 0

    lax.fori_loop(0, BPW // 16, flat_body, 0)

    plsc.subcore_barrier()

    mi_d, mw_d = [], []
    for j in range(NMCH):
        mi_d.append(pltpu.async_copy(
            mif_hbm.at[fidx_v.at[j]], mi_v.at[j], sem_i))
    for j in range(NMCH):
        mw_d.append(pltpu.async_copy(
            mwf_hbm.at[fidx_v.at[j]], mw_v.at[j], sem_w))
    for d in mi_d:
        d.wait()

    mt_d = []
    for j in range(NMCH):
        mt_d.append(pltpu.async_copy(
            sh_tm.at[mi_v.at[j]], meta_v.at[pl.ds(j * CHUNK, CHUNK)], sem_m))
    for d in mw_d:
        d.wait()
    for d in mt_d:
        d.wait()

    def group_body(g, _):
        r = g // 8
        cb = (g % 8) * 16
        wv = [mw_v[t * NIDX + r, pl.ds(cb, 16)] for t in range(T)]
        for l in range(16):
            b = g * 16 + l
            w = [jnp.full((16,), wv[t][l]) for t in range(T)]
            for h in range(D // 16):
                sl = pl.ds(h * 16, 16)
                acc = w[0] * meta_v[b, sl]
                for t in range(1, T):
                    acc = acc + w[t] * meta_v[t * BPW + b, sl]
                ws_v[pl.ds(b * D + h * 16, 16)] = acc
        return 0

    lax.fori_loop(0, BPW // 16, group_body, 0)

    def tr_body(k, _):
        bl = k * 16 + lax.iota(jnp.int32, 16)
        p0 = sid * (CHUNK * D) + bl * D
        for d in range(D):
            tidx_v[d, pl.ds(k * 16, 16)] = p0 + d
        return 0

    lax.fori_loop(0, CHUNK // 16, tr_body, 0)

    for p2 in range(NIDX):
        pltpu.sync_copy(ws_v.at[pl.ds(p2 * CHUNK * D, CHUNK * D)],
                        sh_ws.at[pl.ds(sid * (CHUNK * D), CHUNK * D)])
        tr_d = []
        for d in range(D):
            tr_d.append(pltpu.async_copy(
                sh_ws.at[tidx_v.at[d]],
                wsT_v.at[d, pl.ds(p2 * CHUNK, CHUNK)], sem_i))
        for dd in tr_d:
            dd.wait()

    o_d = []
    for d in range(D):
        o_d.append(pltpu.async_copy(
            wsT_v.at[d], ws_hbm.at[d, pl.ds(base, BPW)], sem_w))
    for dd in o_d:
        dd.wait()


@functools.partial(
    pl.kernel,
    mesh=_mesh,
    out_type=jax.ShapeDtypeStruct((D, B), jnp.float32),
    compiler_params=pltpu.CompilerParams(use_tc_tiling_on_sc=False),
    scratch_types=[
        pltpu.VMEM((NIDX, CHUNK), jnp.int32),
        pltpu.VMEM((D * NIDX, CHUNK), jnp.int32),
        pltpu.VMEM((D, BPW), jnp.float32),
        pltpu.VMEM((D, BPW), jnp.float32),
        pltpu.SemaphoreType.DMA,
        pltpu.SemaphoreType.DMA,
        pltpu.SemaphoreType.DMA,
    ],
)
def _sc_elem_add(e2_hbm, tef_hbm, wsT_hbm, out_hbm,
                 idx_v, fidx_v, el_v, ws_v, sem_e, sem_s, sem_o):
    wid = lax.axis_index("s") * NC + lax.axis_index("c")
    base = wid * BPW

    pltpu.sync_copy(e2_hbm.at[pl.ds(wid * NIDX, NIDX)], idx_v)
    ws_d = []
    for d in range(D):
        ws_d.append(pltpu.async_copy(
            wsT_hbm.at[d, pl.ds(base, BPW)], ws_v.at[d], sem_s))

    def flat_body(k, _):
        v = idx_v[k // 8, pl.ds((k % 8) * 16, 16)]
        for d in range(D):
            fidx_v[d * NIDX + k // 8, pl.ds((k % 8) * 16, 16)] = v + d * NE
        return 0

    lax.fori_loop(0, BPW // 16, flat_body, 0)

    el_d = []
    for d in range(D):
        for c in range(NIDX):
            el_d.append(pltpu.async_copy(
                tef_hbm.at[fidx_v.at[d * NIDX + c]],
                el_v.at[d, pl.ds(c * CHUNK, CHUNK)], sem_e))
    for dd in el_d:
        dd.wait()
    for dd in ws_d:
        dd.wait()

    scale = jnp.float32(0.2)

    def comb_body(r, _):
        for h in range(8):
            sl = pl.ds(128 * (r % NIDX) + 16 * h, 16)
            d = r // NIDX
            ws_v[d, sl] = (el_v[d, sl] + ws_v[d, sl]) * scale
        return 0

    lax.fori_loop(0, D * NIDX, comb_body, 0)

    out_d = []
    for d in range(D):
        out_d.append(pltpu.async_copy(
            ws_v.at[d], out_hbm.at[d, pl.ds(base, BPW)], sem_o))
    for dd in out_d:
        dd.wait()


def kernel(element_indicies, table_element, table_meta, meta_indicies,
           meta_weights):
    e2 = element_indicies.reshape(B // CHUNK, CHUNK)
    wsum = _sc_meta_sum(e2, table_meta,
                        meta_indicies.T.reshape(-1),
                        meta_weights.T.reshape(-1))
    outT = _sc_elem_add(e2, table_element.T.reshape(-1), wsum)
    return outT.T
